# baseline (speedup 1.0000x reference)
import functools

import jax
import jax.numpy as jnp
from jax.experimental import pallas as pl
from jax.experimental.pallas import tpu as pltpu

_MIB = 1024 * 1024


def _se_kernel(x_ref, w1_ref, b1_ref, w2_ref, b2_ref, o_ref, *, inv_hw, c, rows):
    for j in range(x_ref.shape[0]):
        x = x_ref[j]
        x3 = x.reshape(c, rows, 128)
        s = jnp.sum(x3.astype(jnp.float32), axis=(1, 2), keepdims=True)
        pooled = s[:, :, 0] * inv_hw
        h = jnp.dot(w1_ref[...], pooled, preferred_element_type=jnp.float32)
        h = jnp.maximum(h + b1_ref[...], 0.0)
        g = jnp.dot(w2_ref[...], h, preferred_element_type=jnp.float32)
        g = jax.nn.sigmoid(g + b2_ref[...])
        o_ref[j] = (x3 * g[:, :, None].astype(x.dtype)).reshape(c * rows, 128)


def _se_kernel_3d(x_ref, w1t_ref, b1_ref, w2t_ref, b2_ref, o_ref, *, inv_hw):
    x = x_ref[...]
    pooled = jnp.sum(x.astype(jnp.float32), axis=-1) * inv_hw
    h = jnp.dot(pooled, w1t_ref[...], preferred_element_type=jnp.float32)
    h = jnp.maximum(h + b1_ref[...], 0.0)
    g = jnp.dot(h, w2t_ref[...], preferred_element_type=jnp.float32)
    g = jax.nn.sigmoid(g + b2_ref[...])
    o_ref[...] = x * g.astype(x.dtype)[:, :, None]


def kernel(x, w1, b1, w2, b2):
    B, C, H, W = x.shape
    Cr = w1.shape[0]
    HW = H * W
    inv_hw = 1.0 / HW
    w1f = w1.astype(jnp.float32)
    w2f = w2.astype(jnp.float32)

    if HW % 128 == 0:
        rows = HW // 128
        xv = x.reshape(B, C * rows, 128)
        out = pl.pallas_call(
            functools.partial(_se_kernel, inv_hw=inv_hw, c=C, rows=rows),
            out_shape=jax.ShapeDtypeStruct((B, C * rows, 128), x.dtype),
            grid=(B,),
            in_specs=[
                pl.BlockSpec((1, C * rows, 128), lambda i: (i, 0, 0)),
                pl.BlockSpec((Cr, C), lambda i: (0, 0)),
                pl.BlockSpec((Cr, 1), lambda i: (0, 0)),
                pl.BlockSpec((C, Cr), lambda i: (0, 0)),
                pl.BlockSpec((C, 1), lambda i: (0, 0)),
            ],
            out_specs=pl.BlockSpec((1, C * rows, 128), lambda i: (i, 0, 0)),
            compiler_params=pltpu.CompilerParams(
                dimension_semantics=("parallel",),
                vmem_limit_bytes=60 * _MIB),
        )(xv, w1f, b1.astype(jnp.float32).reshape(Cr, 1),
          w2f, b2.astype(jnp.float32).reshape(C, 1))
        return out.reshape(B, C, H, W)

    x_flat = x.reshape(B, C, HW)
    out = pl.pallas_call(
        functools.partial(_se_kernel_3d, inv_hw=inv_hw),
        out_shape=jax.ShapeDtypeStruct((B, C, HW), x.dtype),
        grid=(B,),
        in_specs=[
            pl.BlockSpec((1, C, HW), lambda i: (i, 0, 0)),
            pl.BlockSpec((C, Cr), lambda i: (0, 0)),
            pl.BlockSpec((1, Cr), lambda i: (0, 0)),
            pl.BlockSpec((Cr, C), lambda i: (0, 0)),
            pl.BlockSpec((1, C), lambda i: (0, 0)),
        ],
        out_specs=pl.BlockSpec((1, C, HW), lambda i: (i, 0, 0)),
        compiler_params=pltpu.CompilerParams(
            dimension_semantics=("parallel",),
            vmem_limit_bytes=60 * _MIB),
    )(x_flat, w1f.T, b1.astype(jnp.float32).reshape(1, Cr),
      w2f.T, b2.astype(jnp.float32).reshape(1, C))
    return out.reshape(B, C, H, W)


# speedup vs baseline: 8.0450x; 8.0450x over previous
import functools

import jax
import jax.numpy as jnp
from jax.experimental import pallas as pl
from jax.experimental.pallas import tpu as pltpu

_MIB = 1024 * 1024


def _se_kernel(x_ref, w1t_ref, b1_ref, w2t_ref, b2_ref, o_ref, *, inv_hw):
    x = x_ref[...]
    pooled = jnp.sum(x.astype(jnp.float32), axis=1) * inv_hw
    h = jnp.dot(pooled, w1t_ref[...], preferred_element_type=jnp.float32)
    h = jnp.maximum(h + b1_ref[...], 0.0)
    g = jnp.dot(h, w2t_ref[...], preferred_element_type=jnp.float32)
    g = jax.nn.sigmoid(g + b2_ref[...])
    o_ref[...] = x * g.astype(x.dtype)[:, None, :]


def kernel(x, w1, b1, w2, b2):
    B, C, H, W = x.shape
    Cr = w1.shape[0]
    HW = H * W

    xt = x.transpose(0, 2, 3, 1).reshape(B, HW, C)

    out = pl.pallas_call(
        functools.partial(_se_kernel, inv_hw=1.0 / HW),
        out_shape=jax.ShapeDtypeStruct((B, HW, C), x.dtype),
        grid=(B,),
        in_specs=[
            pl.BlockSpec((1, HW, C), lambda i: (i, 0, 0)),
            pl.BlockSpec((C, Cr), lambda i: (0, 0)),
            pl.BlockSpec((1, Cr), lambda i: (0, 0)),
            pl.BlockSpec((Cr, C), lambda i: (0, 0)),
            pl.BlockSpec((1, C), lambda i: (0, 0)),
        ],
        out_specs=pl.BlockSpec((1, HW, C), lambda i: (i, 0, 0)),
        compiler_params=pltpu.CompilerParams(
            dimension_semantics=("parallel",),
            vmem_limit_bytes=60 * _MIB),
    )(xt, w1.astype(jnp.float32).T, b1.astype(jnp.float32).reshape(1, Cr),
      w2.astype(jnp.float32).T, b2.astype(jnp.float32).reshape(1, C))
    return out.reshape(B, H, W, C).transpose(0, 3, 1, 2)


# speedup vs baseline: 8.0686x; 1.0029x over previous
import functools

import jax
import jax.numpy as jnp
from jax.experimental import pallas as pl
from jax.experimental.pallas import tpu as pltpu

_MIB = 1024 * 1024


_CONTRACT_LAST = (((1,), (1,)), ((), ()))


def _se_kernel(x_ref, w1_ref, b1_ref, w2_ref, b2_ref, o_ref, *, inv_hw):
    x = x_ref[...]
    pooled = jnp.sum(x.astype(jnp.float32), axis=1) * inv_hw
    h = jax.lax.dot_general(pooled, w1_ref[...], _CONTRACT_LAST,
                            preferred_element_type=jnp.float32)
    h = jnp.maximum(h + b1_ref[...], 0.0)
    g = jax.lax.dot_general(h, w2_ref[...], _CONTRACT_LAST,
                            preferred_element_type=jnp.float32)
    g = jax.nn.sigmoid(g + b2_ref[...])
    o_ref[...] = x * g.astype(x.dtype)[:, None, :]


def kernel(x, w1, b1, w2, b2):
    B, C, H, W = x.shape
    Cr = w1.shape[0]
    HW = H * W

    xt = x.transpose(0, 2, 3, 1).reshape(B, HW, C)

    out = pl.pallas_call(
        functools.partial(_se_kernel, inv_hw=1.0 / HW),
        out_shape=jax.ShapeDtypeStruct((B, HW, C), x.dtype),
        grid=(B,),
        in_specs=[
            pl.BlockSpec((1, HW, C), lambda i: (i, 0, 0)),
            pl.BlockSpec((Cr, C), lambda i: (0, 0)),
            pl.BlockSpec((1, Cr), lambda i: (0, 0)),
            pl.BlockSpec((C, Cr), lambda i: (0, 0)),
            pl.BlockSpec((1, C), lambda i: (0, 0)),
        ],
        out_specs=pl.BlockSpec((1, HW, C), lambda i: (i, 0, 0)),
        compiler_params=pltpu.CompilerParams(
            dimension_semantics=("parallel",),
            vmem_limit_bytes=60 * _MIB),
    )(xt, w1.astype(jnp.float32), b1.astype(jnp.float32).reshape(1, Cr),
      w2.astype(jnp.float32), b2.astype(jnp.float32).reshape(1, C))
    return out.reshape(B, H, W, C).transpose(0, 3, 1, 2)


# speedup vs baseline: 8.3565x; 1.0357x over previous
import functools

import jax
import jax.numpy as jnp
from jax.experimental import pallas as pl
from jax.experimental.pallas import tpu as pltpu

_MIB = 1024 * 1024


_CONTRACT_LAST = (((1,), (1,)), ((), ()))


def _se_kernel(x_ref, w1_ref, b1_ref, w2_ref, b2_ref, o_ref, *, inv_hw):
    x = x_ref[...]
    pooled = jnp.sum(x.astype(jnp.float32), axis=1) * inv_hw
    h = jax.lax.dot_general(pooled, w1_ref[...], _CONTRACT_LAST,
                            preferred_element_type=jnp.float32)
    h = jnp.maximum(h + b1_ref[...], 0.0)
    g = jax.lax.dot_general(h, w2_ref[...], _CONTRACT_LAST,
                            preferred_element_type=jnp.float32)
    g = jax.nn.sigmoid(g + b2_ref[...])
    o_ref[...] = x * g.astype(x.dtype)[:, None, :]


def kernel(x, w1, b1, w2, b2):
    B, C, H, W = x.shape
    Cr = w1.shape[0]
    HW = H * W

    xt = x.transpose(0, 2, 3, 1).reshape(B, HW, C)

    slab = C * HW * jnp.dtype(x.dtype).itemsize
    bblk = 1
    for d in (2,):
        if B % d == 0 and B // d >= 2 and 4 * d * slab + 8 * _MIB <= 56 * _MIB:
            bblk = d
    nb = B // bblk

    out = pl.pallas_call(
        functools.partial(_se_kernel, inv_hw=1.0 / HW),
        out_shape=jax.ShapeDtypeStruct((B, HW, C), x.dtype),
        grid=(nb,),
        in_specs=[
            pl.BlockSpec((bblk, HW, C), lambda i: (i, 0, 0)),
            pl.BlockSpec((Cr, C), lambda i: (0, 0)),
            pl.BlockSpec((1, Cr), lambda i: (0, 0)),
            pl.BlockSpec((C, Cr), lambda i: (0, 0)),
            pl.BlockSpec((1, C), lambda i: (0, 0)),
        ],
        out_specs=pl.BlockSpec((bblk, HW, C), lambda i: (i, 0, 0)),
        compiler_params=pltpu.CompilerParams(
            dimension_semantics=("parallel",),
            vmem_limit_bytes=60 * _MIB),
    )(xt, w1.astype(jnp.float32), b1.astype(jnp.float32).reshape(1, Cr),
      w2.astype(jnp.float32), b2.astype(jnp.float32).reshape(1, C))
    return out.reshape(B, H, W, C).transpose(0, 3, 1, 2)


# speedup vs baseline: 8.3780x; 1.0026x over previous
import functools

import jax
import jax.numpy as jnp
from jax.experimental import pallas as pl
from jax.experimental.pallas import tpu as pltpu

_MIB = 1024 * 1024


_CONTRACT_LAST = (((1,), (1,)), ((), ()))


def _se_kernel(x_ref, w1_ref, b1_ref, w2_ref, b2_ref, o_ref, *, inv_hw):
    x = x_ref[...]
    pooled = jnp.sum(x.astype(jnp.float32), axis=1) * inv_hw
    h = jax.lax.dot_general(pooled, w1_ref[...], _CONTRACT_LAST,
                            preferred_element_type=jnp.float32)
    h = jnp.maximum(h + b1_ref[...][None, :], 0.0)
    g = jax.lax.dot_general(h, w2_ref[...], _CONTRACT_LAST,
                            preferred_element_type=jnp.float32)
    g = jax.nn.sigmoid(g + b2_ref[...][None, :])
    o_ref[...] = x * g.astype(x.dtype)[:, None, :]


def kernel(x, w1, b1, w2, b2):
    B, C, H, W = x.shape
    Cr = w1.shape[0]
    HW = H * W

    xt = x.transpose(0, 2, 3, 1).reshape(B, HW, C)

    slab = C * HW * jnp.dtype(x.dtype).itemsize
    bblk = 1
    for d in (2,):
        if B % d == 0 and B // d >= 2 and 4 * d * slab + 8 * _MIB <= 56 * _MIB:
            bblk = d
    nb = B // bblk

    out = pl.pallas_call(
        functools.partial(_se_kernel, inv_hw=1.0 / HW),
        out_shape=jax.ShapeDtypeStruct((B, HW, C), x.dtype),
        grid=(nb,),
        in_specs=[
            pl.BlockSpec((bblk, HW, C), lambda i: (i, 0, 0)),
            pl.BlockSpec((Cr, C), lambda i: (0, 0)),
            pl.BlockSpec((Cr,), lambda i: (0,)),
            pl.BlockSpec((C, Cr), lambda i: (0, 0)),
            pl.BlockSpec((C,), lambda i: (0,)),
        ],
        out_specs=pl.BlockSpec((bblk, HW, C), lambda i: (i, 0, 0)),
        compiler_params=pltpu.CompilerParams(
            dimension_semantics=("parallel",),
            vmem_limit_bytes=60 * _MIB),
    )(xt, w1.astype(jnp.float32), b1.astype(jnp.float32),
      w2.astype(jnp.float32), b2.astype(jnp.float32))
    return out.reshape(B, H, W, C).transpose(0, 3, 1, 2)
